# baseline (speedup 1.0000x reference)
import functools

import jax
import jax.numpy as jnp
from jax import lax
from jax.experimental import pallas as pl
from jax.experimental.pallas import tpu as pltpu
from jax.experimental.pallas import tpu_sc as plsc

_K = 8
_SR = 24
_RADIUS2 = 0.16
_QB = 192
_GW = 128
_TW = 48
_NW = 32
_NS = 16


def _knn_body(q_ref, qsq_ref, ptst_ref, ptsq_ref, idx_ref, d2_ref):
    q = q_ref[...]
    ptst = ptst_ref[...]
    qsq = qsq_ref[...]
    dot = jnp.dot(q, ptst, preferred_element_type=jnp.float32)
    d2 = qsq - 2.0 * dot + ptsq_ref[...]
    n = d2.shape[1]
    quarter = n // 4

    iota = jax.lax.broadcasted_iota(jnp.int32, (d2.shape[0], quarter), 1)
    v = [d2[:, k * quarter:(k + 1) * quarter] for k in range(4)]
    ix = [iota + k * quarter for k in range(4)]

    def ce(a, b, lex=False):
        if lex:
            c = (v[a] < v[b]) | ((v[a] == v[b]) & (ix[a] < ix[b]))
        else:
            c = v[a] <= v[b]
        va, vb = jnp.where(c, v[a], v[b]), jnp.where(c, v[b], v[a])
        ia, ib = jnp.where(c, ix[a], ix[b]), jnp.where(c, ix[b], ix[a])
        v[a], v[b], ix[a], ix[b] = va, vb, ia, ib

    ce(0, 1)
    ce(2, 3)
    ce(0, 2)
    ce(1, 3)
    ce(1, 2, lex=True)

    for j in range(_K):
        m = jnp.min(v[0], axis=1, keepdims=True)
        am = jnp.min(jnp.where(v[0] == m, ix[0], n), axis=1, keepdims=True)
        d2_ref[:, j:j + 1] = m
        idx_ref[:, j:j + 1] = am
        if j + 1 < _K:
            hit = ix[0] == am
            v[0] = jnp.where(hit, v[1], v[0])
            ix[0] = jnp.where(hit, ix[1], ix[0])
            v[1] = jnp.where(hit, v[2], v[1])
            ix[1] = jnp.where(hit, ix[2], ix[1])
            v[2] = jnp.where(hit, v[3], v[2])
            ix[2] = jnp.where(hit, ix[3], ix[2])
            v[3] = jnp.where(hit, jnp.inf, v[3])


def _knn(q, qsq, ptst, ptsq):
    nq = q.shape[0]
    n = ptst.shape[1]
    grid = nq // _QB
    return pl.pallas_call(
        _knn_body,
        grid=(grid,),
        in_specs=[
            pl.BlockSpec((_QB, 3), lambda i: (i, 0)),
            pl.BlockSpec((_QB, 1), lambda i: (i, 0)),
            pl.BlockSpec((3, n), lambda i: (0, 0)),
            pl.BlockSpec((1, n), lambda i: (0, 0)),
        ],
        out_specs=[
            pl.BlockSpec((_QB, _K), lambda i: (i, 0)),
            pl.BlockSpec((_QB, _K), lambda i: (i, 0)),
        ],
        out_shape=[
            jax.ShapeDtypeStruct((nq, _K), jnp.int32),
            jax.ShapeDtypeStruct((nq, _K), jnp.float32),
        ],
    )(q, qsq, ptst, ptsq)


def _sc_gather(table, idx):
    n_idx = idx.shape[0]
    per_w = n_idx // _NW

    @functools.partial(
        pl.kernel,
        out_type=jax.ShapeDtypeStruct((n_idx, _TW), table.dtype),
        mesh=plsc.VectorSubcoreMesh(core_axis_name="c", subcore_axis_name="s"),
        compiler_params=pltpu.CompilerParams(use_tc_tiling_on_sc=False),
        scratch_types=[
            pltpu.VMEM((per_w,), jnp.int32),
            pltpu.VMEM((per_w, _TW), jnp.float32),
            pltpu.SemaphoreType.DMA,
        ],
    )
    def k(table_hbm, idx_hbm, out_hbm, idx_v, rows_v, sem):
        wid = lax.axis_index("s") * 2 + lax.axis_index("c")
        base = wid * per_w
        pltpu.sync_copy(idx_hbm.at[pl.ds(base, per_w)], idx_v)
        pltpu.async_copy(table_hbm.at[idx_v], rows_v, sem).wait()
        pltpu.sync_copy(rows_v, out_hbm.at[pl.ds(base, per_w)])

    return k(table, idx)


def kernel(point_cloud_pos, points_embeddings, points_color, points_dir,
           raydir, camrotc2w, campos, near, far):
    rd = raydir[0]
    r = rd.shape[0]
    t = jnp.linspace(near[0], far[0], _SR)
    raypos = campos[0][None, None, :] + rd[:, None, :] * t[None, :, None]
    q = raypos.reshape(-1, 3)
    nq = q.shape[0]
    n = point_cloud_pos.shape[0]

    qsq = jnp.sum(q * q, axis=-1, keepdims=True)
    ptsq = jnp.sum(point_cloud_pos * point_cloud_pos, axis=-1)[None, :]
    ptst = point_cloud_pos.T

    d = points_embeddings.shape[1]
    pad = _TW - (3 + d + 3 + 3)
    table = jnp.concatenate(
        [point_cloud_pos, points_embeddings, points_color, points_dir,
         jnp.zeros((n, pad), jnp.float32)], axis=1).astype(jnp.float32)

    slab = nq // _NS
    pidx_slabs, gath_slabs = [], []
    for s in range(_NS):
        lo = s * slab
        idx_s, d2_s = _knn(q[lo:lo + slab], qsq[lo:lo + slab], ptst, ptsq)
        spidx = jnp.where(d2_s <= _RADIUS2, idx_s, -1)
        pidx_slabs.append(spidx)
        gath_slabs.append(_sc_gather(table, jnp.maximum(spidx, 0).reshape(-1)))

    sample_pidx = jnp.concatenate(pidx_slabs, axis=0)
    sample_pnt_mask = (sample_pidx >= 0).reshape(1, r, _SR, _K)
    g = jnp.concatenate(gath_slabs, axis=0)

    sampled_xyz = g[:, :3].reshape(1, r, _SR, _K, 3)
    sampled_embedding = g[:, 3:3 + d].reshape(1, r, _SR, _K, d)
    sampled_color = g[:, 3 + d:6 + d].reshape(1, r, _SR, _K, 3)
    sampled_dir = g[:, 6 + d:9 + d].reshape(1, r, _SR, _K, 3)

    sample_loc_cam_coor = ((raypos - campos[0][None, None, :]) @ camrotc2w[0])[None]
    sample_ray_dirs = jnp.broadcast_to(rd[:, None, :], (r, _SR, 3))[None]
    return (sampled_color, sampled_dir, sampled_embedding, sampled_xyz,
            sample_pnt_mask.reshape(1, r, _SR, _K), raypos[None],
            sample_loc_cam_coor, sample_ray_dirs)


# speedup vs baseline: 1.0544x; 1.0544x over previous
import functools

import jax
import jax.numpy as jnp
from jax import lax
from jax.experimental import pallas as pl
from jax.experimental.pallas import tpu as pltpu
from jax.experimental.pallas import tpu_sc as plsc

_K = 8
_SR = 24
_RADIUS2 = 0.16
_QB = 128
_GW = 128
_TW = 48
_NW = 32
_NS = 16


def _knn_body(q_ref, qsq_ref, ptst_ref, ptsq_ref, idx_ref, d2_ref):
    q = q_ref[...]
    ptst = ptst_ref[...]
    qsq = qsq_ref[...]
    dot = jnp.dot(q, ptst, preferred_element_type=jnp.float32)
    d2 = qsq - 2.0 * dot + ptsq_ref[...]
    n = d2.shape[1]
    quarter = n // 4

    iota = jax.lax.broadcasted_iota(jnp.int32, (d2.shape[0], quarter), 1)
    v = [d2[:, k * quarter:(k + 1) * quarter] for k in range(4)]
    ix = [iota + k * quarter for k in range(4)]

    def ce(a, b, lex=False):
        if lex:
            c = (v[a] < v[b]) | ((v[a] == v[b]) & (ix[a] < ix[b]))
        else:
            c = v[a] <= v[b]
        va, vb = jnp.where(c, v[a], v[b]), jnp.where(c, v[b], v[a])
        ia, ib = jnp.where(c, ix[a], ix[b]), jnp.where(c, ix[b], ix[a])
        v[a], v[b], ix[a], ix[b] = va, vb, ia, ib

    ce(0, 1)
    ce(2, 3)
    ce(0, 2)
    ce(1, 3)
    ce(1, 2, lex=True)

    for j in range(_K):
        m = jnp.min(v[0], axis=1, keepdims=True)
        am = jnp.min(jnp.where(v[0] == m, ix[0], n), axis=1, keepdims=True)
        d2_ref[:, j:j + 1] = m
        idx_ref[:, j:j + 1] = am
        if j + 1 < _K:
            hit = ix[0] == am
            v[0] = jnp.where(hit, v[1], v[0])
            ix[0] = jnp.where(hit, ix[1], ix[0])
            v[1] = jnp.where(hit, v[2], v[1])
            ix[1] = jnp.where(hit, ix[2], ix[1])
            v[2] = jnp.where(hit, v[3], v[2])
            ix[2] = jnp.where(hit, ix[3], ix[2])
            v[3] = jnp.where(hit, jnp.inf, v[3])


def _knn(q, qsq, ptst, ptsq):
    nq = q.shape[0]
    n = ptst.shape[1]
    grid = nq // _QB
    return pl.pallas_call(
        _knn_body,
        grid=(grid,),
        in_specs=[
            pl.BlockSpec((_QB, 3), lambda i: (i, 0)),
            pl.BlockSpec((_QB, 1), lambda i: (i, 0)),
            pl.BlockSpec((3, n), lambda i: (0, 0)),
            pl.BlockSpec((1, n), lambda i: (0, 0)),
        ],
        out_specs=[
            pl.BlockSpec((_QB, _K), lambda i: (i, 0)),
            pl.BlockSpec((_QB, _K), lambda i: (i, 0)),
        ],
        out_shape=[
            jax.ShapeDtypeStruct((nq, _K), jnp.int32),
            jax.ShapeDtypeStruct((nq, _K), jnp.float32),
        ],
    )(q, qsq, ptst, ptsq)


def _sc_gather(table, idx):
    n_idx = idx.shape[0]
    per_w = n_idx // _NW

    @functools.partial(
        pl.kernel,
        out_type=jax.ShapeDtypeStruct((n_idx, _TW), table.dtype),
        mesh=plsc.VectorSubcoreMesh(core_axis_name="c", subcore_axis_name="s"),
        compiler_params=pltpu.CompilerParams(use_tc_tiling_on_sc=False),
        scratch_types=[
            pltpu.VMEM((per_w,), jnp.int32),
            pltpu.VMEM((per_w, _TW), jnp.float32),
            pltpu.SemaphoreType.DMA,
        ],
    )
    def k(table_hbm, idx_hbm, out_hbm, idx_v, rows_v, sem):
        wid = lax.axis_index("s") * 2 + lax.axis_index("c")
        base = wid * per_w
        pltpu.sync_copy(idx_hbm.at[pl.ds(base, per_w)], idx_v)
        pltpu.async_copy(table_hbm.at[idx_v], rows_v, sem).wait()
        pltpu.sync_copy(rows_v, out_hbm.at[pl.ds(base, per_w)])

    return k(table, idx)


def kernel(point_cloud_pos, points_embeddings, points_color, points_dir,
           raydir, camrotc2w, campos, near, far):
    rd = raydir[0]
    r = rd.shape[0]
    t = jnp.linspace(near[0], far[0], _SR)
    raypos = campos[0][None, None, :] + rd[:, None, :] * t[None, :, None]
    q = raypos.reshape(-1, 3)
    nq = q.shape[0]
    n = point_cloud_pos.shape[0]

    qsq = jnp.sum(q * q, axis=-1, keepdims=True)
    ptsq = jnp.sum(point_cloud_pos * point_cloud_pos, axis=-1)[None, :]
    ptst = point_cloud_pos.T

    d = points_embeddings.shape[1]
    pad = _TW - (3 + d + 3 + 3)
    table = jnp.concatenate(
        [point_cloud_pos, points_embeddings, points_color, points_dir,
         jnp.zeros((n, pad), jnp.float32)], axis=1).astype(jnp.float32)

    slab = nq // _NS
    pidx_slabs, gath_slabs = [], []
    for s in range(_NS):
        lo = s * slab
        idx_s, d2_s = _knn(q[lo:lo + slab], qsq[lo:lo + slab], ptst, ptsq)
        spidx = jnp.where(d2_s <= _RADIUS2, idx_s, -1)
        pidx_slabs.append(spidx)
        gath_slabs.append(_sc_gather(table, jnp.maximum(spidx, 0).reshape(-1)))

    sample_pidx = jnp.concatenate(pidx_slabs, axis=0)
    sample_pnt_mask = (sample_pidx >= 0).reshape(1, r, _SR, _K)
    g = jnp.concatenate(gath_slabs, axis=0)

    sampled_xyz = g[:, :3].reshape(1, r, _SR, _K, 3)
    sampled_embedding = g[:, 3:3 + d].reshape(1, r, _SR, _K, d)
    sampled_color = g[:, 3 + d:6 + d].reshape(1, r, _SR, _K, 3)
    sampled_dir = g[:, 6 + d:9 + d].reshape(1, r, _SR, _K, 3)

    sample_loc_cam_coor = ((raypos - campos[0][None, None, :]) @ camrotc2w[0])[None]
    sample_ray_dirs = jnp.broadcast_to(rd[:, None, :], (r, _SR, 3))[None]
    return (sampled_color, sampled_dir, sampled_embedding, sampled_xyz,
            sample_pnt_mask.reshape(1, r, _SR, _K), raypos[None],
            sample_loc_cam_coor, sample_ray_dirs)
